# Initial kernel scaffold; baseline (speedup 1.0000x reference)
import jax
import jax.numpy as jnp
from jax.experimental import pallas as pl


def kernel(q, k, v, Wq, bq, Wk, bk, Wv, bv, Wo, bo):
    raise NotImplementedError("write your pallas kernel here")


# speedup vs baseline: 1.4525x; 1.4525x over previous
import jax
import jax.numpy as jnp
from jax.experimental import pallas as pl

B, T, D, H = 2, 2048, 1024, 16
DH = D // H
TOP_K = 8


def _mm_kernel(x_ref, w_ref, b_ref, o_ref):
    o_ref[...] = jnp.dot(x_ref[...], w_ref[...],
                         preferred_element_type=jnp.float32) + b_ref[...]


def _mm(x, w, b):
    n = x.shape[0]
    blk = 512
    return pl.pallas_call(
        _mm_kernel,
        grid=(n // blk,),
        in_specs=[
            pl.BlockSpec((blk, x.shape[1]), lambda i: (i, 0)),
            pl.BlockSpec((w.shape[0], w.shape[1]), lambda i: (0, 0)),
            pl.BlockSpec((w.shape[1],), lambda i: (0,)),
        ],
        out_specs=pl.BlockSpec((blk, w.shape[1]), lambda i: (i, 0)),
        out_shape=jax.ShapeDtypeStruct((n, w.shape[1]), jnp.float32),
    )(x, w, b)


def kernel(q, k, v, Wq, bq, Wk, bk, Wv, bv, Wo, bo):
    qp = q @ Wq + bq
    kp = k @ Wk + bk

    qt = qp.transpose(0, 2, 1)
    kt = kp.transpose(0, 2, 1)

    q_fft = jnp.fft.rfft(qt, axis=-1)
    k_fft = jnp.fft.rfft(kt, axis=-1)
    R = jnp.fft.irfft(q_fft * jnp.conj(k_fft), n=T, axis=-1)

    weights, indices = jax.lax.top_k(R, TOP_K)
    corr = jax.nn.softmax(weights, axis=-1)

    tmp_values = jnp.concatenate([qt, qt], axis=-1)
    init_index = jnp.arange(T, dtype=indices.dtype)[None, None, :]

    agg = jnp.zeros_like(qt)
    for i in range(TOP_K):
        idx = init_index + indices[..., i:i + 1]
        pattern = jnp.take_along_axis(tmp_values, idx, axis=-1)
        agg = agg + pattern * corr[..., i:i + 1]

    x = agg.transpose(0, 2, 1).reshape(B * T, D)
    out = _mm(x, Wo, bo)
    return out.reshape(B, T, D)


# speedup vs baseline: 2.4047x; 1.6555x over previous
import functools

import jax
import jax.numpy as jnp
from jax import lax
from jax.experimental import pallas as pl
from jax.experimental.pallas import tpu as pltpu
from jax.experimental.pallas import tpu_sc as plsc

B, T, D, H = 2, 2048, 1024, 16
DH = D // H
TOP_K = 8

NC, NS = 2, 16
NW = NC * NS
ROWS = B * D
RPW = ROWS // NW


def _mm_kernel(x_ref, w_ref, b_ref, o_ref):
    o_ref[...] = jnp.dot(x_ref[...], w_ref[...],
                         preferred_element_type=jnp.float32) + b_ref[...]


def _mm(x, w, b):
    n = x.shape[0]
    blk = 512
    return pl.pallas_call(
        _mm_kernel,
        grid=(n // blk,),
        in_specs=[
            pl.BlockSpec((blk, x.shape[1]), lambda i: (i, 0)),
            pl.BlockSpec((w.shape[0], w.shape[1]), lambda i: (0, 0)),
            pl.BlockSpec((w.shape[1],), lambda i: (0,)),
        ],
        out_specs=pl.BlockSpec((blk, w.shape[1]), lambda i: (i, 0)),
        out_shape=jax.ShapeDtypeStruct((n, w.shape[1]), jnp.float32),
    )(x, w, b)


def _agg_body(qt2, idxh, wh, outh, idx_s, w_s, bufs, acc, sem_s, sem_in, sem_out):
    wid = lax.axis_index("s") * NC + lax.axis_index("c")
    base = wid * RPW
    pltpu.async_copy(idxh.at[pl.ds(base, RPW)], idx_s, sem_s).wait()
    pltpu.async_copy(wh.at[pl.ds(base, RPW)], w_s, sem_s).wait()

    @pl.loop(0, RPW)
    def _(r):
        row = base + r
        tau_vec = idx_s[r, pl.ds(0, 16)]
        w_vec = w_s[r, pl.ds(0, 16)]
        copies = []
        rems = []
        for i in range(TOP_K):
            tau = tau_vec[i]
            rem = lax.rem(tau, 8)
            tau0 = pl.multiple_of(tau - rem, 8)
            rems.append(rem)
            copies.append(
                pltpu.async_copy(qt2.at[row, pl.ds(tau0, T + 8)], bufs.at[i],
                                 sem_in))
        for cp in copies:
            cp.wait()
        ws = [w_vec[i] for i in range(TOP_K)]

        @pl.loop(0, T, step=16)
        def _(c):
            a = bufs[0, pl.ds(c + rems[0], 16)] * ws[0]
            for i in range(1, TOP_K):
                a += bufs[i, pl.ds(c + rems[i], 16)] * ws[i]
            acc[pl.ds(c, 16)] = a

        pltpu.async_copy(acc, outh.at[row], sem_out).wait()


@jax.jit
def _agg(qt2, idx2, w2):
    mesh = plsc.VectorSubcoreMesh(core_axis_name="c", subcore_axis_name="s")
    kfn = pl.kernel(
        _agg_body,
        out_type=jax.ShapeDtypeStruct((ROWS, T), jnp.float32),
        mesh=mesh,
        compiler_params=pltpu.CompilerParams(use_tc_tiling_on_sc=False),
        scratch_types=[
            pltpu.VMEM((RPW, 16), jnp.int32),
            pltpu.VMEM((RPW, 16), jnp.float32),
            pltpu.VMEM((TOP_K, T + 8), jnp.float32),
            pltpu.VMEM((T,), jnp.float32),
            pltpu.SemaphoreType.DMA,
            pltpu.SemaphoreType.DMA,
            pltpu.SemaphoreType.DMA,
        ],
    )
    return kfn(qt2, idx2, w2)


def kernel(q, k, v, Wq, bq, Wk, bk, Wv, bv, Wo, bo):
    qp = q @ Wq + bq
    kp = k @ Wk + bk

    qt = qp.transpose(0, 2, 1)
    kt = kp.transpose(0, 2, 1)

    q_fft = jnp.fft.rfft(qt, axis=-1)
    k_fft = jnp.fft.rfft(kt, axis=-1)
    R = jnp.fft.irfft(q_fft * jnp.conj(k_fft), n=T, axis=-1)

    weights, indices = jax.lax.top_k(R, TOP_K)
    corr = jax.nn.softmax(weights, axis=-1)

    qt2 = jnp.concatenate([qt, qt], axis=-1).reshape(ROWS, 2 * T)
    pad = ((0, 0), (0, 16 - TOP_K))
    idx2 = jnp.pad(indices.reshape(ROWS, TOP_K), pad)
    w2 = jnp.pad(corr.reshape(ROWS, TOP_K), pad)

    agg = _agg(qt2, idx2, w2).reshape(B, D, T)

    x = agg.transpose(0, 2, 1).reshape(B * T, D)
    out = _mm(x, Wo, bo)
    return out.reshape(B, T, D)


# speedup vs baseline: 5.7353x; 2.3850x over previous
import functools

import numpy as np
import jax
import jax.numpy as jnp
from jax import lax
from jax.experimental import pallas as pl
from jax.experimental.pallas import tpu as pltpu
from jax.experimental.pallas import tpu_sc as plsc
import ml_dtypes

B, T, D, H = 2, 2048, 1024, 16
DH = D // H
TOP_K = 8
FP = 1152

NC, NS = 2, 16
NW = NC * NS
ROWS = B * D
RPW = ROWS // NW

_BF = ml_dtypes.bfloat16


def _np_split3(x):
    x = x.astype(np.float32)
    hi = x.astype(_BF)
    r1 = x - hi.astype(np.float32)
    lo = r1.astype(_BF)
    lo2 = (r1 - lo.astype(np.float32)).astype(_BF)
    return hi, lo, lo2


def _make_tables():
    t = np.arange(T, dtype=np.float64)
    f = np.arange(FP, dtype=np.float64)
    ang = 2.0 * np.pi * np.outer(t, f) / T
    cf = np.cos(ang).astype(np.float32)
    sf = np.sin(ang).astype(np.float32)
    w = np.zeros(FP, dtype=np.float64)
    w[1:1024] = 2.0 / T
    w[0] = 1.0 / T
    w[1024] = 1.0 / T
    angi = 2.0 * np.pi * np.outer(f, t) / T
    ci = (w[:, None] * np.cos(angi)).astype(np.float32)
    si = (w[:, None] * np.sin(angi)).astype(np.float32)
    return (_np_split3(cf), _np_split3(sf), _np_split3(ci), _np_split3(si))


_CF3, _SF3, _CI3, _SI3 = _make_tables()


def _split_f32(x):
    hi = x.astype(jnp.bfloat16)
    lo = (x - hi.astype(jnp.float32)).astype(jnp.bfloat16)
    return hi, lo


def _split3_f32(x):
    hi = x.astype(jnp.bfloat16)
    r1 = x - hi.astype(jnp.float32)
    lo = r1.astype(jnp.bfloat16)
    lo2 = (r1 - lo.astype(jnp.float32)).astype(jnp.bfloat16)
    return hi, lo, lo2


def _dot(a, b, dims):
    return lax.dot_general(a, b, (dims, ((), ())),
                           preferred_element_type=jnp.float32)


def _mm2(ah, al, bh, bl, dims):
    return _dot(ah, bh, dims) + _dot(ah, bl, dims) + _dot(al, bh, dims)


def _mm6(a3, b3, dims):
    ah, al, al2 = a3
    bh, bl, bl2 = b3
    small = (_dot(ah, bl2, dims) + _dot(al, bl, dims) + _dot(al2, bh, dims))
    mid = _dot(ah, bl, dims) + _dot(al, bh, dims)
    return small + mid + _dot(ah, bh, dims)


_TBA = 512


def _proj_body(q_ref, k_ref, wq_ref, bq_ref, wk_ref, bk_ref, qt_ref, kt_ref):
    cdims = ((1,), (1,))
    qh = q_ref[0].astype(jnp.bfloat16)
    qt_ref[0] = _dot(wq_ref[...], qh, cdims) + bq_ref[...][:, None]
    kh = k_ref[0].astype(jnp.bfloat16)
    kt_ref[0] = _dot(wk_ref[...], kh, cdims) + bk_ref[...][:, None]


def _proj(q, k, Wq, bq, Wk, bk):
    full = lambda s, d: pl.BlockSpec(s, lambda b, t: tuple(0 for _ in s))
    return pl.pallas_call(
        _proj_body,
        grid=(B, T // _TBA),
        in_specs=[
            pl.BlockSpec((1, _TBA, D), lambda b, t: (b, t, 0)),
            pl.BlockSpec((1, _TBA, D), lambda b, t: (b, t, 0)),
            full((D, D), None), full((D,), None),
            full((D, D), None), full((D,), None),
        ],
        out_specs=[
            pl.BlockSpec((1, D, _TBA), lambda b, t: (b, 0, t)),
            pl.BlockSpec((1, D, _TBA), lambda b, t: (b, 0, t)),
        ],
        out_shape=[
            jax.ShapeDtypeStruct((B, D, T), jnp.float32),
            jax.ShapeDtypeStruct((B, D, T), jnp.float32),
        ],
    )(q, k, Wq.T.astype(jnp.bfloat16), bq, Wk.T.astype(jnp.bfloat16), bk)


_CB = 128


def _fwd_body(qt_ref, kt_ref, cfh_ref, cfl_ref, cfl2_ref, sfh_ref, sfl_ref,
              sfl2_ref, pr_ref, pi_ref):
    cdims = ((1,), (0,))
    q3 = _split3_f32(qt_ref[0])
    k3 = _split3_f32(kt_ref[0])
    cf3 = (cfh_ref[...], cfl_ref[...], cfl2_ref[...])
    sf3 = (sfh_ref[...], sfl_ref[...], sfl2_ref[...])
    qc = _mm6(q3, cf3, cdims)
    qs = _mm6(q3, sf3, cdims)
    kc = _mm6(k3, cf3, cdims)
    ks = _mm6(k3, sf3, cdims)
    pr_ref[0] = qc * kc + qs * ks
    pi_ref[0] = qc * ks - qs * kc


def _fwd(qt, kt):
    full = lambda s: pl.BlockSpec(s, lambda b, c: tuple(0 for _ in s))
    return pl.pallas_call(
        _fwd_body,
        grid=(B, D // _CB),
        in_specs=[
            pl.BlockSpec((1, _CB, T), lambda b, c: (b, c, 0)),
            pl.BlockSpec((1, _CB, T), lambda b, c: (b, c, 0)),
            full((T, FP)), full((T, FP)), full((T, FP)),
            full((T, FP)), full((T, FP)), full((T, FP)),
        ],
        out_specs=[
            pl.BlockSpec((1, _CB, FP), lambda b, c: (b, c, 0)),
            pl.BlockSpec((1, _CB, FP), lambda b, c: (b, c, 0)),
        ],
        out_shape=[
            jax.ShapeDtypeStruct((B, D, FP), jnp.float32),
            jax.ShapeDtypeStruct((B, D, FP), jnp.float32),
        ],
    )(qt, kt, *(jnp.asarray(a) for a in _CF3),
      *(jnp.asarray(a) for a in _SF3))


def _inv_body(pr_ref, pi_ref, cih_ref, cil_ref, cil2_ref, sih_ref, sil_ref,
              sil2_ref, r_ref):
    cdims = ((1,), (0,))
    p3 = _split3_f32(pr_ref[0])
    i3 = _split3_f32(pi_ref[0])
    ci3 = (cih_ref[...], cil_ref[...], cil2_ref[...])
    si3 = (sih_ref[...], sil_ref[...], sil2_ref[...])
    r_ref[0] = _mm6(p3, ci3, cdims) - _mm6(i3, si3, cdims)


def _inv(pr, pi):
    full = lambda s: pl.BlockSpec(s, lambda b, c: tuple(0 for _ in s))
    return pl.pallas_call(
        _inv_body,
        grid=(B, D // _CB),
        in_specs=[
            pl.BlockSpec((1, _CB, FP), lambda b, c: (b, c, 0)),
            pl.BlockSpec((1, _CB, FP), lambda b, c: (b, c, 0)),
            full((FP, T)), full((FP, T)), full((FP, T)),
            full((FP, T)), full((FP, T)), full((FP, T)),
        ],
        out_specs=pl.BlockSpec((1, _CB, T), lambda b, c: (b, c, 0)),
        out_shape=jax.ShapeDtypeStruct((B, D, T), jnp.float32),
    )(pr, pi, *(jnp.asarray(a) for a in _CI3),
      *(jnp.asarray(a) for a in _SI3))


_CG = 64


def _topk_body(r_ref, idx_ref, w_ref):
    iota = lax.broadcasted_iota(jnp.int32, (8, T), 1)
    for g in range(_CG // 8):
        x = r_ref[0, pl.ds(g * 8, 8), :]
        vals = []
        idxs = []
        for _ in range(TOP_K):
            m = jnp.max(x, axis=1, keepdims=True)
            am = jnp.min(jnp.where(x == m, iota, T), axis=1, keepdims=True)
            vals.append(m)
            idxs.append(am)
            x = jnp.where(iota == am, -jnp.inf, x)
        v = jnp.concatenate(vals, axis=1)
        ix = jnp.concatenate(idxs, axis=1)
        e = jnp.exp(v - v[:, 0:1])
        w = e / jnp.sum(e, axis=1, keepdims=True)
        zi = jnp.zeros((8, 16 - TOP_K), jnp.int32)
        zw = jnp.zeros((8, 16 - TOP_K), jnp.float32)
        idx_ref[0, pl.ds(g * 8, 8), :] = jnp.concatenate([ix, zi], axis=1)
        w_ref[0, pl.ds(g * 8, 8), :] = jnp.concatenate([w, zw], axis=1)


def _topk(r):
    return pl.pallas_call(
        _topk_body,
        grid=(B, D // _CG),
        in_specs=[pl.BlockSpec((1, _CG, T), lambda b, c: (b, c, 0))],
        out_specs=[
            pl.BlockSpec((1, _CG, 16), lambda b, c: (b, c, 0)),
            pl.BlockSpec((1, _CG, 16), lambda b, c: (b, c, 0)),
        ],
        out_shape=[
            jax.ShapeDtypeStruct((B, D, 16), jnp.int32),
            jax.ShapeDtypeStruct((B, D, 16), jnp.float32),
        ],
    )(r)


def _agg_body(qt2, idxh, wh, outh, idx_s, w_s, bufs, acc, sem_s, sem_in,
              sem_out):
    wid = lax.axis_index("s") * NC + lax.axis_index("c")
    base = wid * RPW
    pltpu.async_copy(idxh.at[pl.ds(base, RPW)], idx_s, sem_s).wait()
    pltpu.async_copy(wh.at[pl.ds(base, RPW)], w_s, sem_s).wait()

    @pl.loop(0, RPW)
    def _(r):
        row = base + r
        tau_vec = idx_s[r, pl.ds(0, 16)]
        w_vec = w_s[r, pl.ds(0, 16)]
        copies = []
        rems = []
        for i in range(TOP_K):
            tau = tau_vec[i]
            rem = lax.rem(tau, 8)
            tau0 = pl.multiple_of(tau - rem, 8)
            rems.append(rem)
            copies.append(
                pltpu.async_copy(qt2.at[row, pl.ds(tau0, T + 8)], bufs.at[i],
                                 sem_in))
        for cp in copies:
            cp.wait()
        ws = [w_vec[i] for i in range(TOP_K)]

        @pl.loop(0, T, step=16)
        def _(c):
            a = bufs[0, pl.ds(c + rems[0], 16)] * ws[0]
            for i in range(1, TOP_K):
                a += bufs[i, pl.ds(c + rems[i], 16)] * ws[i]
            acc[pl.ds(c, 16)] = a

        pltpu.async_copy(acc, outh.at[row], sem_out).wait()


def _agg(qt2, idx2, w2):
    mesh = plsc.VectorSubcoreMesh(core_axis_name="c", subcore_axis_name="s")
    kfn = pl.kernel(
        _agg_body,
        out_type=jax.ShapeDtypeStruct((ROWS, T), jnp.float32),
        mesh=mesh,
        compiler_params=pltpu.CompilerParams(use_tc_tiling_on_sc=False),
        scratch_types=[
            pltpu.VMEM((RPW, 16), jnp.int32),
            pltpu.VMEM((RPW, 16), jnp.float32),
            pltpu.VMEM((TOP_K, T + 8), jnp.float32),
            pltpu.VMEM((T,), jnp.float32),
            pltpu.SemaphoreType.DMA,
            pltpu.SemaphoreType.DMA,
            pltpu.SemaphoreType.DMA,
        ],
    )
    return kfn(qt2, idx2, w2)


_TBO = 512


def _out_body(agg_ref, wo_ref, bo_ref, o_ref):
    ah = agg_ref[0].astype(jnp.bfloat16)
    o_ref[0] = (_dot(ah, wo_ref[...], ((0,), (0,)))
                + bo_ref[...][None, :])


def _outproj(agg, Wo, bo):
    full = lambda s: pl.BlockSpec(s, lambda b, t: tuple(0 for _ in s))
    return pl.pallas_call(
        _out_body,
        grid=(B, T // _TBO),
        in_specs=[
            pl.BlockSpec((1, D, _TBO), lambda b, t: (b, 0, t)),
            full((D, D)), full((D,)),
        ],
        out_specs=pl.BlockSpec((1, _TBO, D), lambda b, t: (b, t, 0)),
        out_shape=jax.ShapeDtypeStruct((B, T, D), jnp.float32),
    )(agg, Wo.astype(jnp.bfloat16), bo)


def kernel(q, k, v, Wq, bq, Wk, bk, Wv, bv, Wo, bo):
    qt, kt = _proj(q, k, Wq, bq, Wk, bk)
    pr, pi = _fwd(qt, kt)
    r = _inv(pr, pi)
    idx, w = _topk(r)

    qt2 = jnp.concatenate([qt, qt], axis=-1).reshape(ROWS, 2 * T)
    agg = _agg(qt2, idx.reshape(ROWS, 16), w.reshape(ROWS, 16))
    return _outproj(agg.reshape(B, D, T), Wo, bo)


# speedup vs baseline: 7.5382x; 1.3144x over previous
import functools

import numpy as np
import jax
import jax.numpy as jnp
from jax import lax
from jax.experimental import pallas as pl
from jax.experimental.pallas import tpu as pltpu
from jax.experimental.pallas import tpu_sc as plsc
import ml_dtypes

B, T, D, H = 2, 2048, 1024, 16
DH = D // H
TOP_K = 8
FP = 1152

NC, NS = 2, 16
NW = NC * NS
ROWS = B * D
RPW = ROWS // NW

_BF = ml_dtypes.bfloat16


def _np_split3(x):
    x = x.astype(np.float32)
    hi = x.astype(_BF)
    r1 = x - hi.astype(np.float32)
    lo = r1.astype(_BF)
    lo2 = (r1 - lo.astype(np.float32)).astype(_BF)
    return hi, lo, lo2


def _make_tables():
    t = np.arange(T, dtype=np.float64)
    f = np.arange(FP, dtype=np.float64)
    ang = 2.0 * np.pi * np.outer(t, f) / T
    cf = np.cos(ang).astype(np.float32)
    sf = np.sin(ang).astype(np.float32)
    w = np.zeros(FP, dtype=np.float64)
    w[1:1024] = 2.0 / T
    w[0] = 1.0 / T
    w[1024] = 1.0 / T
    angi = 2.0 * np.pi * np.outer(f, t) / T
    ci = (w[:, None] * np.cos(angi)).astype(np.float32)
    si = (w[:, None] * np.sin(angi)).astype(np.float32)
    return (_np_split3(cf), _np_split3(sf), _np_split3(ci), _np_split3(si))


_CF3, _SF3, _CI3, _SI3 = _make_tables()


def _split_f32(x):
    hi = x.astype(jnp.bfloat16)
    lo = (x - hi.astype(jnp.float32)).astype(jnp.bfloat16)
    return hi, lo


def _split3_f32(x):
    hi = x.astype(jnp.bfloat16)
    r1 = x - hi.astype(jnp.float32)
    lo = r1.astype(jnp.bfloat16)
    lo2 = (r1 - lo.astype(jnp.float32)).astype(jnp.bfloat16)
    return hi, lo, lo2


def _dot(a, b, dims):
    return lax.dot_general(a, b, (dims, ((), ())),
                           preferred_element_type=jnp.float32)


def _mm2(ah, al, bh, bl, dims):
    return _dot(ah, bh, dims) + _dot(ah, bl, dims) + _dot(al, bh, dims)


def _mm6(a3, b3, dims):
    ah, al, al2 = a3
    bh, bl, bl2 = b3
    small = (_dot(ah, bl2, dims) + _dot(al, bl, dims) + _dot(al2, bh, dims))
    mid = _dot(ah, bl, dims) + _dot(al, bh, dims)
    return small + mid + _dot(ah, bh, dims)


_TBA = 512


def _proj_body(q_ref, k_ref, wq_ref, bq_ref, wk_ref, bk_ref, qt_ref, kt_ref):
    cdims = ((1,), (1,))
    qh = q_ref[0].astype(jnp.bfloat16)
    qt_ref[0] = _dot(wq_ref[...], qh, cdims) + bq_ref[...][:, None]
    kh = k_ref[0].astype(jnp.bfloat16)
    kt_ref[0] = _dot(wk_ref[...], kh, cdims) + bk_ref[...][:, None]


def _proj(q, k, Wq, bq, Wk, bk):
    full = lambda s, d: pl.BlockSpec(s, lambda b, t: tuple(0 for _ in s))
    return pl.pallas_call(
        _proj_body,
        grid=(B, T // _TBA),
        in_specs=[
            pl.BlockSpec((1, _TBA, D), lambda b, t: (b, t, 0)),
            pl.BlockSpec((1, _TBA, D), lambda b, t: (b, t, 0)),
            full((D, D), None), full((D,), None),
            full((D, D), None), full((D,), None),
        ],
        out_specs=[
            pl.BlockSpec((1, D, _TBA), lambda b, t: (b, 0, t)),
            pl.BlockSpec((1, D, _TBA), lambda b, t: (b, 0, t)),
        ],
        out_shape=[
            jax.ShapeDtypeStruct((B, D, T), jnp.float32),
            jax.ShapeDtypeStruct((B, D, T), jnp.float32),
        ],
    )(q, k, Wq.T.astype(jnp.bfloat16), bq, Wk.T.astype(jnp.bfloat16), bk)


_CB = 128


def _fwd_body(qt_ref, kt_ref, cfh_ref, cfl_ref, sfh_ref, sfl_ref,
              pr_ref, pi_ref):
    cdims = ((1,), (0,))
    qh, ql = _split_f32(qt_ref[0])
    kh, kl = _split_f32(kt_ref[0])
    cfh, cfl = cfh_ref[...], cfl_ref[...]
    sfh, sfl = sfh_ref[...], sfl_ref[...]
    qc = _mm2(qh, ql, cfh, cfl, cdims)
    qs = _mm2(qh, ql, sfh, sfl, cdims)
    kc = _mm2(kh, kl, cfh, cfl, cdims)
    ks = _mm2(kh, kl, sfh, sfl, cdims)
    pr_ref[0] = qc * kc + qs * ks
    pi_ref[0] = qc * ks - qs * kc


def _fwd(qt, kt):
    full = lambda s: pl.BlockSpec(s, lambda b, c: tuple(0 for _ in s))
    return pl.pallas_call(
        _fwd_body,
        grid=(B, D // _CB),
        in_specs=[
            pl.BlockSpec((1, _CB, T), lambda b, c: (b, c, 0)),
            pl.BlockSpec((1, _CB, T), lambda b, c: (b, c, 0)),
            full((T, FP)), full((T, FP)), full((T, FP)), full((T, FP)),
        ],
        out_specs=[
            pl.BlockSpec((1, _CB, FP), lambda b, c: (b, c, 0)),
            pl.BlockSpec((1, _CB, FP), lambda b, c: (b, c, 0)),
        ],
        out_shape=[
            jax.ShapeDtypeStruct((B, D, FP), jnp.float32),
            jax.ShapeDtypeStruct((B, D, FP), jnp.float32),
        ],
    )(qt, kt, jnp.asarray(_CF3[0]), jnp.asarray(_CF3[1]),
      jnp.asarray(_SF3[0]), jnp.asarray(_SF3[1]))


def _inv_body(pr_ref, pi_ref, cih_ref, cil_ref, sih_ref, sil_ref, r_ref):
    cdims = ((1,), (0,))
    prh, prl = _split_f32(pr_ref[0])
    pih, pil = _split_f32(pi_ref[0])
    r_ref[0] = (_mm2(prh, prl, cih_ref[...], cil_ref[...], cdims)
                - _mm2(pih, pil, sih_ref[...], sil_ref[...], cdims))


def _inv(pr, pi):
    full = lambda s: pl.BlockSpec(s, lambda b, c: tuple(0 for _ in s))
    return pl.pallas_call(
        _inv_body,
        grid=(B, D // _CB),
        in_specs=[
            pl.BlockSpec((1, _CB, FP), lambda b, c: (b, c, 0)),
            pl.BlockSpec((1, _CB, FP), lambda b, c: (b, c, 0)),
            full((FP, T)), full((FP, T)), full((FP, T)), full((FP, T)),
        ],
        out_specs=pl.BlockSpec((1, _CB, T), lambda b, c: (b, c, 0)),
        out_shape=jax.ShapeDtypeStruct((B, D, T), jnp.float32),
    )(pr, pi, jnp.asarray(_CI3[0]), jnp.asarray(_CI3[1]),
      jnp.asarray(_SI3[0]), jnp.asarray(_SI3[1]))


_CG = 64


def _topk_body(r_ref, idx_ref, w_ref):
    iota = lax.broadcasted_iota(jnp.int32, (8, T), 1)
    for g in range(_CG // 8):
        x = r_ref[0, pl.ds(g * 8, 8), :]
        vals = []
        idxs = []
        for _ in range(TOP_K):
            m = jnp.max(x, axis=1, keepdims=True)
            am = jnp.min(jnp.where(x == m, iota, T), axis=1, keepdims=True)
            vals.append(m)
            idxs.append(am)
            x = jnp.where(iota == am, -jnp.inf, x)
        v = jnp.concatenate(vals, axis=1)
        ix = jnp.concatenate(idxs, axis=1)
        e = jnp.exp(v - v[:, 0:1])
        w = e / jnp.sum(e, axis=1, keepdims=True)
        zi = jnp.zeros((8, 16 - TOP_K), jnp.int32)
        zw = jnp.zeros((8, 16 - TOP_K), jnp.float32)
        idx_ref[0, pl.ds(g * 8, 8), :] = jnp.concatenate([ix, zi], axis=1)
        w_ref[0, pl.ds(g * 8, 8), :] = jnp.concatenate([w, zw], axis=1)


def _topk(r):
    return pl.pallas_call(
        _topk_body,
        grid=(B, D // _CG),
        in_specs=[pl.BlockSpec((1, _CG, T), lambda b, c: (b, c, 0))],
        out_specs=[
            pl.BlockSpec((1, _CG, 16), lambda b, c: (b, c, 0)),
            pl.BlockSpec((1, _CG, 16), lambda b, c: (b, c, 0)),
        ],
        out_shape=[
            jax.ShapeDtypeStruct((B, D, 16), jnp.int32),
            jax.ShapeDtypeStruct((B, D, 16), jnp.float32),
        ],
    )(r)


def _agg_body(qt2, idxh, wh, outh, idx_s, w_s, bufs, acc, sem_s, sem_in,
              sem_out):
    wid = lax.axis_index("s") * NC + lax.axis_index("c")
    base = wid * RPW
    pltpu.async_copy(idxh.at[pl.ds(base, RPW)], idx_s, sem_s).wait()
    pltpu.async_copy(wh.at[pl.ds(base, RPW)], w_s, sem_s).wait()

    @pl.loop(0, RPW)
    def _(r):
        row = base + r
        tau_vec = idx_s[r, pl.ds(0, 16)]
        w_vec = w_s[r, pl.ds(0, 16)]
        copies = []
        rems = []
        for i in range(TOP_K):
            tau = tau_vec[i]
            rem = lax.rem(tau, 8)
            tau0 = pl.multiple_of(tau - rem, 8)
            rems.append(rem)
            copies.append(
                pltpu.async_copy(qt2.at[row, pl.ds(tau0, T + 8)], bufs.at[i],
                                 sem_in))
        for cp in copies:
            cp.wait()
        ws = [w_vec[i] for i in range(TOP_K)]

        @pl.loop(0, T, step=16)
        def _(c):
            a = bufs[0, pl.ds(c + rems[0], 16)] * ws[0]
            for i in range(1, TOP_K):
                a += bufs[i, pl.ds(c + rems[i], 16)] * ws[i]
            acc[pl.ds(c, 16)] = a

        pltpu.async_copy(acc, outh.at[row], sem_out).wait()


def _agg(qt2, idx2, w2):
    mesh = plsc.VectorSubcoreMesh(core_axis_name="c", subcore_axis_name="s")
    kfn = pl.kernel(
        _agg_body,
        out_type=jax.ShapeDtypeStruct((ROWS, T), jnp.float32),
        mesh=mesh,
        compiler_params=pltpu.CompilerParams(use_tc_tiling_on_sc=False),
        scratch_types=[
            pltpu.VMEM((RPW, 16), jnp.int32),
            pltpu.VMEM((RPW, 16), jnp.float32),
            pltpu.VMEM((TOP_K, T + 8), jnp.float32),
            pltpu.VMEM((T,), jnp.float32),
            pltpu.SemaphoreType.DMA,
            pltpu.SemaphoreType.DMA,
            pltpu.SemaphoreType.DMA,
        ],
    )
    return kfn(qt2, idx2, w2)


_TBO = 512


def _out_body(agg_ref, wo_ref, bo_ref, o_ref):
    ah = agg_ref[0].astype(jnp.bfloat16)
    o_ref[0] = (_dot(ah, wo_ref[...], ((0,), (0,)))
                + bo_ref[...][None, :])


def _outproj(agg, Wo, bo):
    full = lambda s: pl.BlockSpec(s, lambda b, t: tuple(0 for _ in s))
    return pl.pallas_call(
        _out_body,
        grid=(B, T // _TBO),
        in_specs=[
            pl.BlockSpec((1, D, _TBO), lambda b, t: (b, 0, t)),
            full((D, D)), full((D,)),
        ],
        out_specs=pl.BlockSpec((1, _TBO, D), lambda b, t: (b, t, 0)),
        out_shape=jax.ShapeDtypeStruct((B, T, D), jnp.float32),
    )(agg, Wo.astype(jnp.bfloat16), bo)


def kernel(q, k, v, Wq, bq, Wk, bk, Wv, bv, Wo, bo):
    qt, kt = _proj(q, k, Wq, bq, Wk, bk)
    pr, pi = _fwd(qt, kt)
    r = _inv(pr, pi)
    idx, w = _topk(r)

    qt2 = jnp.concatenate([qt, qt], axis=-1).reshape(ROWS, 2 * T)
    agg = _agg(qt2, idx.reshape(ROWS, 16), w.reshape(ROWS, 16))
    return _outproj(agg.reshape(B, D, T), Wo, bo)


# speedup vs baseline: 9.3464x; 1.2399x over previous
import functools

import numpy as np
import jax
import jax.numpy as jnp
from jax import lax
from jax.experimental import pallas as pl
from jax.experimental.pallas import tpu as pltpu
from jax.experimental.pallas import tpu_sc as plsc
import ml_dtypes

B, T, D, H = 2, 2048, 1024, 16
DH = D // H
TOP_K = 8
FP = 1152

NC, NS = 2, 16
NW = NC * NS
ROWS = B * D
RPW = ROWS // NW

_BF = ml_dtypes.bfloat16


def _np_split3(x):
    x = x.astype(np.float32)
    hi = x.astype(_BF)
    r1 = x - hi.astype(np.float32)
    lo = r1.astype(_BF)
    lo2 = (r1 - lo.astype(np.float32)).astype(_BF)
    return hi, lo, lo2


def _make_tables():
    t = np.arange(T, dtype=np.float64)
    f = np.arange(FP, dtype=np.float64)
    ang = 2.0 * np.pi * np.outer(t, f) / T
    cf = np.cos(ang).astype(np.float32)
    sf = np.sin(ang).astype(np.float32)
    w = np.zeros(FP, dtype=np.float64)
    w[1:1024] = 2.0 / T
    w[0] = 1.0 / T
    w[1024] = 1.0 / T
    angi = 2.0 * np.pi * np.outer(f, t) / T
    ci = (w[:, None] * np.cos(angi)).astype(np.float32)
    si = (w[:, None] * np.sin(angi)).astype(np.float32)
    return (_np_split3(cf), _np_split3(sf), _np_split3(ci), _np_split3(si))


_CF3, _SF3, _CI3, _SI3 = _make_tables()


def _split_f32(x):
    hi = x.astype(jnp.bfloat16)
    lo = (x - hi.astype(jnp.float32)).astype(jnp.bfloat16)
    return hi, lo


def _split3_f32(x):
    hi = x.astype(jnp.bfloat16)
    r1 = x - hi.astype(jnp.float32)
    lo = r1.astype(jnp.bfloat16)
    lo2 = (r1 - lo.astype(jnp.float32)).astype(jnp.bfloat16)
    return hi, lo, lo2


def _dot(a, b, dims):
    return lax.dot_general(a, b, (dims, ((), ())),
                           preferred_element_type=jnp.float32)


def _mm2(ah, al, bh, bl, dims):
    return _dot(ah, bh, dims) + _dot(ah, bl, dims) + _dot(al, bh, dims)


def _mm6(a3, b3, dims):
    ah, al, al2 = a3
    bh, bl, bl2 = b3
    small = (_dot(ah, bl2, dims) + _dot(al, bl, dims) + _dot(al2, bh, dims))
    mid = _dot(ah, bl, dims) + _dot(al, bh, dims)
    return small + mid + _dot(ah, bh, dims)


_TBA = 512


def _proj_body(q_ref, k_ref, wq_ref, bq_ref, wk_ref, bk_ref, qt_ref, kt_ref):
    cdims = ((1,), (1,))
    qh = q_ref[0].astype(jnp.bfloat16)
    qt_ref[0] = _dot(wq_ref[...], qh, cdims) + bq_ref[...][:, None]
    kh = k_ref[0].astype(jnp.bfloat16)
    kt_ref[0] = _dot(wk_ref[...], kh, cdims) + bk_ref[...][:, None]


def _proj(q, k, Wq, bq, Wk, bk):
    full = lambda s, d: pl.BlockSpec(s, lambda b, t: tuple(0 for _ in s))
    return pl.pallas_call(
        _proj_body,
        grid=(B, T // _TBA),
        in_specs=[
            pl.BlockSpec((1, _TBA, D), lambda b, t: (b, t, 0)),
            pl.BlockSpec((1, _TBA, D), lambda b, t: (b, t, 0)),
            full((D, D), None), full((D,), None),
            full((D, D), None), full((D,), None),
        ],
        out_specs=[
            pl.BlockSpec((1, D, _TBA), lambda b, t: (b, 0, t)),
            pl.BlockSpec((1, D, _TBA), lambda b, t: (b, 0, t)),
        ],
        out_shape=[
            jax.ShapeDtypeStruct((B, D, T), jnp.float32),
            jax.ShapeDtypeStruct((B, D, T), jnp.float32),
        ],
    )(q, k, Wq.T.astype(jnp.bfloat16), bq, Wk.T.astype(jnp.bfloat16), bk)


_CB = 128


def _fwd_body(qt_ref, kt_ref, cfh_ref, cfl_ref, sfh_ref, sfl_ref,
              pr_ref, pi_ref):
    cdims = ((1,), (0,))
    qh, ql = _split_f32(qt_ref[0])
    kh, kl = _split_f32(kt_ref[0])
    cfh, cfl = cfh_ref[...], cfl_ref[...]
    sfh, sfl = sfh_ref[...], sfl_ref[...]
    qc = _mm2(qh, ql, cfh, cfl, cdims)
    qs = _mm2(qh, ql, sfh, sfl, cdims)
    kc = _mm2(kh, kl, cfh, cfl, cdims)
    ks = _mm2(kh, kl, sfh, sfl, cdims)
    pr_ref[0] = qc * kc + qs * ks
    pi_ref[0] = qc * ks - qs * kc


def _fwd(qt, kt):
    full = lambda s: pl.BlockSpec(s, lambda b, c: tuple(0 for _ in s))
    return pl.pallas_call(
        _fwd_body,
        grid=(qt.shape[0], D // _CB),
        in_specs=[
            pl.BlockSpec((1, _CB, T), lambda b, c: (b, c, 0)),
            pl.BlockSpec((1, _CB, T), lambda b, c: (b, c, 0)),
            full((T, FP)), full((T, FP)), full((T, FP)), full((T, FP)),
        ],
        out_specs=[
            pl.BlockSpec((1, _CB, FP), lambda b, c: (b, c, 0)),
            pl.BlockSpec((1, _CB, FP), lambda b, c: (b, c, 0)),
        ],
        out_shape=[
            jax.ShapeDtypeStruct((qt.shape[0], D, FP), jnp.float32),
            jax.ShapeDtypeStruct((qt.shape[0], D, FP), jnp.float32),
        ],
    )(qt, kt, jnp.asarray(_CF3[0]), jnp.asarray(_CF3[1]),
      jnp.asarray(_SF3[0]), jnp.asarray(_SF3[1]))


def _inv_body(pr_ref, pi_ref, cih_ref, cil_ref, sih_ref, sil_ref, r_ref):
    cdims = ((1,), (0,))
    prh, prl = _split_f32(pr_ref[0])
    pih, pil = _split_f32(pi_ref[0])
    r_ref[0] = (_mm2(prh, prl, cih_ref[...], cil_ref[...], cdims)
                - _mm2(pih, pil, sih_ref[...], sil_ref[...], cdims))


def _inv(pr, pi):
    full = lambda s: pl.BlockSpec(s, lambda b, c: tuple(0 for _ in s))
    return pl.pallas_call(
        _inv_body,
        grid=(pr.shape[0], D // _CB),
        in_specs=[
            pl.BlockSpec((1, _CB, FP), lambda b, c: (b, c, 0)),
            pl.BlockSpec((1, _CB, FP), lambda b, c: (b, c, 0)),
            full((FP, T)), full((FP, T)), full((FP, T)), full((FP, T)),
        ],
        out_specs=pl.BlockSpec((1, _CB, T), lambda b, c: (b, c, 0)),
        out_shape=jax.ShapeDtypeStruct((pr.shape[0], D, T), jnp.float32),
    )(pr, pi, jnp.asarray(_CI3[0]), jnp.asarray(_CI3[1]),
      jnp.asarray(_SI3[0]), jnp.asarray(_SI3[1]))


_CG = 64


def _topk_body(r_ref, idx_ref, w_ref):
    iota = lax.broadcasted_iota(jnp.int32, (8, T), 1)
    for g in range(_CG // 8):
        x = r_ref[0, pl.ds(g * 8, 8), :]
        vals = []
        idxs = []
        for _ in range(TOP_K):
            m = jnp.max(x, axis=1, keepdims=True)
            am = jnp.min(jnp.where(x == m, iota, T), axis=1, keepdims=True)
            vals.append(m)
            idxs.append(am)
            x = jnp.where(iota == am, -jnp.inf, x)
        v = jnp.concatenate(vals, axis=1)
        ix = jnp.concatenate(idxs, axis=1)
        e = jnp.exp(v - v[:, 0:1])
        w = e / jnp.sum(e, axis=1, keepdims=True)
        zi = jnp.zeros((8, 16 - TOP_K), jnp.int32)
        zw = jnp.zeros((8, 16 - TOP_K), jnp.float32)
        idx_ref[0, pl.ds(g * 8, 8), :] = jnp.concatenate([ix, zi], axis=1)
        w_ref[0, pl.ds(g * 8, 8), :] = jnp.concatenate([w, zw], axis=1)


def _topk(r):
    return pl.pallas_call(
        _topk_body,
        grid=(r.shape[0], D // _CG),
        in_specs=[pl.BlockSpec((1, _CG, T), lambda b, c: (b, c, 0))],
        out_specs=[
            pl.BlockSpec((1, _CG, 16), lambda b, c: (b, c, 0)),
            pl.BlockSpec((1, _CG, 16), lambda b, c: (b, c, 0)),
        ],
        out_shape=[
            jax.ShapeDtypeStruct((r.shape[0], D, 16), jnp.int32),
            jax.ShapeDtypeStruct((r.shape[0], D, 16), jnp.float32),
        ],
    )(r)


def _agg(qt, idx2, w2):
    rows = qt.shape[0]
    rpw = rows // NW

    def body(qth, idxh, wh, outh, idx_s, w_s, bufs, acc,
             sem_s, sem0, sem1, semo0, semo1):
        wid = lax.axis_index("s") * NC + lax.axis_index("c")
        base = wid * rpw
        pltpu.async_copy(idxh.at[pl.ds(base, rpw)], idx_s, sem_s).wait()
        pltpu.async_copy(wh.at[pl.ds(base, rpw)], w_s, sem_s).wait()

        def prefetch(row, p, sem):
            pltpu.async_copy(qth.at[row], bufs.at[p, pl.ds(0, T)], sem)
            pltpu.async_copy(qth.at[row], bufs.at[p, pl.ds(T, T)], sem)

        def wait_in(p, sem):
            pltpu.make_async_copy(qth.at[0], bufs.at[p, pl.ds(0, T)],
                                  sem).wait()
            pltpu.make_async_copy(qth.at[0], bufs.at[p, pl.ds(T, T)],
                                  sem).wait()

        def compute(r, p, semo):
            row = base + r
            tau_vec = idx_s[r, pl.ds(0, 16)]
            w_vec = w_s[r, pl.ds(0, 16)]
            taus = [tau_vec[i] for i in range(TOP_K)]
            ws = [w_vec[i] for i in range(TOP_K)]

            @pl.loop(0, T, step=16)
            def _(c):
                a = bufs[p, pl.ds(taus[0] + c, 16)] * ws[0]
                for i in range(1, TOP_K):
                    a += bufs[p, pl.ds(taus[i] + c, 16)] * ws[i]
                acc[p, pl.ds(c, 16)] = a

            pltpu.async_copy(acc.at[p], outh.at[row], semo)

        prefetch(base, 0, sem0)

        @pl.loop(0, rpw, step=2)
        def _(r):
            prefetch(base + r + 1, 1, sem1)
            wait_in(0, sem0)

            @pl.when(r >= 2)
            def _():
                pltpu.make_async_copy(acc.at[0], outh.at[base], semo0).wait()

            compute(r, 0, semo0)
            prefetch(base + lax.rem(r + 2, rpw), 0, sem0)
            wait_in(1, sem1)

            @pl.when(r >= 2)
            def _():
                pltpu.make_async_copy(acc.at[1], outh.at[base], semo1).wait()

            compute(r + 1, 1, semo1)

        wait_in(0, sem0)
        pltpu.make_async_copy(acc.at[0], outh.at[base], semo0).wait()
        pltpu.make_async_copy(acc.at[1], outh.at[base], semo1).wait()

    mesh = plsc.VectorSubcoreMesh(core_axis_name="c", subcore_axis_name="s")
    kfn = pl.kernel(
        body,
        out_type=jax.ShapeDtypeStruct((rows, T), jnp.float32),
        mesh=mesh,
        compiler_params=pltpu.CompilerParams(use_tc_tiling_on_sc=False),
        scratch_types=[
            pltpu.VMEM((rpw, 16), jnp.int32),
            pltpu.VMEM((rpw, 16), jnp.float32),
            pltpu.VMEM((2, 2 * T), jnp.float32),
            pltpu.VMEM((2, T), jnp.float32),
            pltpu.SemaphoreType.DMA,
            pltpu.SemaphoreType.DMA,
            pltpu.SemaphoreType.DMA,
            pltpu.SemaphoreType.DMA,
            pltpu.SemaphoreType.DMA,
        ],
    )
    return kfn(qt, idx2, w2)


_TBO = 512


def _out_body(agg_ref, wo_ref, bo_ref, o_ref):
    ah = agg_ref[0].astype(jnp.bfloat16)
    o_ref[0] = (_dot(ah, wo_ref[...], ((0,), (0,)))
                + bo_ref[...][None, :])


def _outproj(agg, Wo, bo):
    full = lambda s: pl.BlockSpec(s, lambda b, t: tuple(0 for _ in s))
    return pl.pallas_call(
        _out_body,
        grid=(agg.shape[0], T // _TBO),
        in_specs=[
            pl.BlockSpec((1, D, _TBO), lambda b, t: (b, 0, t)),
            full((D, D)), full((D,)),
        ],
        out_specs=pl.BlockSpec((1, _TBO, D), lambda b, t: (b, t, 0)),
        out_shape=jax.ShapeDtypeStruct((agg.shape[0], T, D), jnp.float32),
    )(agg, Wo.astype(jnp.bfloat16), bo)


def kernel(q, k, v, Wq, bq, Wk, bk, Wv, bv, Wo, bo):
    qt, kt = _proj(q, k, Wq, bq, Wk, bk)
    outs = []
    for b in range(B):
        qtb = qt[b:b + 1]
        pr, pi = _fwd(qtb, kt[b:b + 1])
        r = _inv(pr, pi)
        idx, w = _topk(r)
        agg = _agg(qt[b], idx[0], w[0])
        outs.append(_outproj(agg[None], Wo, bo))
    return jnp.concatenate(outs, axis=0)
